# baseline (speedup 1.0000x reference)
import functools

import jax
import jax.numpy as jnp
from jax.experimental import pallas as pl
from jax.experimental.pallas import tpu as pltpu

BEAM = 4
VOCAB = 32768
PAD_ID = 0
EOS_ID = 2
LPF = 0.6
BATCH = 128

ROWS = BATCH * BEAM
ROW_BLK = 64
RG = 8
LANES = 128
CHUNKS = VOCAB // LANES
NEG = -3.0e38
BIGIDX = 2 ** 30
NSEL = 6


def _sel_kernel(p_ref, stop_ref, wl_ref, wp_ref, score_ref, flat_ref,
                kscr, iscr):
    lane1 = jax.lax.broadcasted_iota(jnp.int32, (1, LANES), 1)
    colbad = (lane1 == PAD_ID) | (lane1 == EOS_ID)

    def row_group(rg, _):
        p = p_ref[pl.ds(rg * RG, RG), :]
        lane = jax.lax.broadcasted_iota(jnp.int32, (RG, LANES), 1)

        def chunk(c, carry):
            v1, v2, v3, v4, i1, i2, i3, i4 = carry
            x = wp_ref[pl.ds(rg * RG, RG), pl.ds(c * LANES, LANES)]
            key = jnp.maximum(p * x, jnp.float32(1e-20))
            key = jnp.where((c == 0) & colbad, jnp.float32(-1.0), key)
            gi = lane + c * LANES
            m = key > v1
            v1, key = jnp.where(m, key, v1), jnp.where(m, v1, key)
            i1, gi = jnp.where(m, gi, i1), jnp.where(m, i1, gi)
            m = key > v2
            v2, key = jnp.where(m, key, v2), jnp.where(m, v2, key)
            i2, gi = jnp.where(m, gi, i2), jnp.where(m, i2, gi)
            m = key > v3
            v3, key = jnp.where(m, key, v3), jnp.where(m, v3, key)
            i3, gi = jnp.where(m, gi, i3), jnp.where(m, i3, gi)
            m = key > v4
            v4 = jnp.where(m, key, v4)
            i4 = jnp.where(m, gi, i4)
            return v1, v2, v3, v4, i1, i2, i3, i4

        zf = jnp.full((RG, LANES), jnp.float32(-2.0))
        zi = jnp.zeros((RG, LANES), jnp.int32)
        v1, v2, v3, v4, i1, i2, i3, i4 = jax.lax.fori_loop(
            0, CHUNKS, chunk, (zf, zf, zf, zf, zi, zi, zi, zi))

        for t in range(NSEL):
            mx = jnp.max(v1, axis=1, keepdims=True)
            eq = v1 == mx
            mi = jnp.min(jnp.where(eq, i1, BIGIDX), axis=1, keepdims=True)
            sel = eq & (i1 == mi)
            kscr[pl.ds(rg * RG, RG), t:t + 1] = mx
            iscr[pl.ds(rg * RG, RG), t:t + 1] = mi
            v1 = jnp.where(sel, v2, v1)
            i1 = jnp.where(sel, i2, i1)
            v2 = jnp.where(sel, v3, v2)
            i2 = jnp.where(sel, i3, i2)
            v3 = jnp.where(sel, v4, v3)
            i3 = jnp.where(sel, i4, i3)
            v4 = jnp.where(sel, NEG, v4)
        return 0

    jax.lax.fori_loop(0, ROW_BLK // RG, row_group, 0)

    p = p_ref[...]
    stopb = stop_ref[...] != 0

    row = jax.lax.broadcasted_iota(jnp.int32, (ROW_BLK, 1), 0)
    beam = row % BEAM
    base = beam * VOCAB

    k = kscr[...]
    vi = iscr[...]
    lane6 = jax.lax.broadcasted_iota(jnp.int32, (ROW_BLK, NSEL), 1)
    floorkey = jnp.maximum(p * 0.0, jnp.float32(1e-20))
    stop_k = jnp.where(lane6 < BEAM, floorkey, 0.0)
    stop_f = jnp.where(lane6 < BEAM, lane6 + 1, BIGIDX + base + lane6)
    sel_k = jnp.where(stopb, stop_k, k)
    sel_f = jnp.where(stopb, stop_f, base + vi)

    w0 = wp_ref[:, PAD_ID:PAD_ID + 1]
    w2 = wp_ref[:, EOS_ID:EOS_ID + 1]
    k0 = jnp.maximum(jnp.where(stopb, p, p * w0), jnp.float32(1e-20))
    f0 = base
    k2u = jnp.maximum(p * w2, jnp.float32(1e-20))
    slot7_k = jnp.where(stopb, 0.0, k2u)
    slot7_f = jnp.where(stopb, BIGIDX + base + 7, base + EOS_ID)

    score_ref[...] = jnp.concatenate([sel_k, k0, slot7_k], axis=1)
    flat_ref[...] = jnp.concatenate([sel_f, f0, slot7_f], axis=1)


def _merge_kernel(score_ref, flat_ref, bs_ref, nw_ref, pi_ref):
    s = score_ref[...]
    f = flat_ref[...]
    batch = jax.lax.broadcasted_iota(jnp.int32, (BATCH, 1), 0)
    bs, nw, pi = [], [], []
    for _ in range(BEAM):
        mx = jnp.max(s, axis=1, keepdims=True)
        eq = s == mx
        mi = jnp.min(jnp.where(eq, f, jnp.int32(2 ** 31 - 1)),
                     axis=1, keepdims=True)
        sel = eq & (f == mi)
        bs.append(mx)
        nw.append(mi % VOCAB)
        pi.append(batch * BEAM + mi // VOCAB)
        s = jnp.where(sel, NEG, s)
    bs_ref[...] = jnp.concatenate(bs, axis=1)
    nw_ref[...] = jnp.concatenate(nw, axis=1)
    pi_ref[...] = jnp.concatenate(pi, axis=1)


@jax.jit
def kernel(word_prob, prob, stops, word_length):
    p2 = prob.reshape(ROWS, 1)
    st2 = stops.reshape(ROWS, 1)
    wl2 = word_length.reshape(ROWS, 1)

    grid = ROWS // ROW_BLK
    keys, flats = pl.pallas_call(
        _sel_kernel,
        grid=(grid,),
        in_specs=[
            pl.BlockSpec((ROW_BLK, 1), lambda i: (i, 0)),
            pl.BlockSpec((ROW_BLK, 1), lambda i: (i, 0)),
            pl.BlockSpec((ROW_BLK, 1), lambda i: (i, 0)),
            pl.BlockSpec((ROW_BLK, VOCAB), lambda i: (i, 0)),
        ],
        out_specs=[
            pl.BlockSpec((ROW_BLK, 8), lambda i: (i, 0)),
            pl.BlockSpec((ROW_BLK, 8), lambda i: (i, 0)),
        ],
        out_shape=[
            jax.ShapeDtypeStruct((ROWS, 8), jnp.float32),
            jax.ShapeDtypeStruct((ROWS, 8), jnp.int32),
        ],
        scratch_shapes=[
            pltpu.VMEM((ROW_BLK, NSEL), jnp.float32),
            pltpu.VMEM((ROW_BLK, NSEL), jnp.int32),
        ],
    )(p2, st2, wl2, word_prob)

    slot_is_sel = (jnp.arange(8, dtype=jnp.int32) < NSEL).astype(jnp.int32)
    addl = slot_is_sel[None, :] * (1 - st2)
    wl_c = wl2 + addl
    lp = (jnp.power((wl_c + 5).astype(jnp.float32), LPF)
          / jnp.power(jnp.float32(6.0), LPF))
    scores = jnp.log(keys) / lp

    sc = scores.reshape(BATCH, BEAM * 8)
    fl = flats.reshape(BATCH, BEAM * 8)
    bs, nw, pi = pl.pallas_call(
        _merge_kernel,
        out_shape=[
            jax.ShapeDtypeStruct((BATCH, BEAM), jnp.float32),
            jax.ShapeDtypeStruct((BATCH, BEAM), jnp.int32),
            jax.ShapeDtypeStruct((BATCH, BEAM), jnp.int32),
        ],
    )(sc, fl)
    return bs, nw.reshape(-1), pi.reshape(-1)


# speedup vs baseline: 12.3673x; 12.3673x over previous
import functools

import jax
import jax.numpy as jnp
from jax.experimental import pallas as pl
from jax.experimental.pallas import tpu as pltpu

BEAM = 4
VOCAB = 32768
PAD_ID = 0
EOS_ID = 2
LPF = 0.6
BATCH = 128

ROWS = BATCH * BEAM
ROW_BLK = 64
RG = 32
LANES = 128
CHUNKS = VOCAB // LANES
NEG = -3.0e38
BIGIDX = 2 ** 30
NSEL = 6


def _sel_kernel(p_ref, stop_ref, wl_ref, wp_ref, score_ref, flat_ref,
                kscr, iscr):
    lane1 = jax.lax.broadcasted_iota(jnp.int32, (1, LANES), 1)
    colbad = (lane1 == PAD_ID) | (lane1 == EOS_ID)

    for rg in range(ROW_BLK // RG):
        p = p_ref[pl.ds(rg * RG, RG), :]
        lane = jax.lax.broadcasted_iota(jnp.int32, (RG, LANES), 1)

        key0 = p * wp_ref[pl.ds(rg * RG, RG), pl.ds(0, LANES)]
        key0 = jnp.where(colbad, jnp.float32(-1.0), key0)
        v1 = key0
        lo = jnp.full((RG, LANES), jnp.float32(-2.0))
        zi = jnp.zeros((RG, LANES), jnp.int32)

        def chunk(c, carry):
            v1, v2, v3, v4, i1, i2, i3, i4 = carry
            x = wp_ref[pl.ds(rg * RG, RG), pl.ds(c * LANES, LANES)]
            key = p * x
            ci = c
            m = key > v1
            v1, key = jnp.where(m, key, v1), jnp.where(m, v1, key)
            i1, ci = jnp.where(m, ci, i1), jnp.where(m, i1, ci)
            m = key > v2
            v2, key = jnp.where(m, key, v2), jnp.where(m, v2, key)
            i2, ci = jnp.where(m, ci, i2), jnp.where(m, i2, ci)
            m = key > v3
            v3, key = jnp.where(m, key, v3), jnp.where(m, v3, key)
            i3, ci = jnp.where(m, ci, i3), jnp.where(m, i3, ci)
            m = key > v4
            v4 = jnp.where(m, key, v4)
            i4 = jnp.where(m, ci, i4)
            return v1, v2, v3, v4, i1, i2, i3, i4

        v1, v2, v3, v4, i1, i2, i3, i4 = jax.lax.fori_loop(
            1, CHUNKS, chunk, (v1, lo, lo, lo, zi, zi, zi, zi),
            unroll=2)

        for t in range(NSEL):
            full1 = i1 * LANES + lane
            mx = jnp.max(v1, axis=1, keepdims=True)
            eq = v1 == mx
            mi = jnp.min(jnp.where(eq, full1, BIGIDX), axis=1, keepdims=True)
            sel = eq & (full1 == mi)
            kscr[pl.ds(rg * RG, RG), t:t + 1] = mx
            iscr[pl.ds(rg * RG, RG), t:t + 1] = mi
            v1 = jnp.where(sel, v2, v1)
            i1 = jnp.where(sel, i2, i1)
            v2 = jnp.where(sel, v3, v2)
            i2 = jnp.where(sel, i3, i2)
            v3 = jnp.where(sel, v4, v3)
            i3 = jnp.where(sel, i4, i3)
            v4 = jnp.where(sel, NEG, v4)

    p = p_ref[...]
    stopb = stop_ref[...] != 0

    row = jax.lax.broadcasted_iota(jnp.int32, (ROW_BLK, 1), 0)
    beam = row % BEAM
    base = beam * VOCAB

    k = kscr[...]
    vi = iscr[...]
    lane6 = jax.lax.broadcasted_iota(jnp.int32, (ROW_BLK, NSEL), 1)
    stop_f = jnp.where(lane6 < BEAM, base + lane6 + 1, BIGIDX + base + lane6)
    sel_k = jnp.where(stopb, 0.0, k)
    sel_f = jnp.where(stopb, stop_f, base + vi)

    w0 = wp_ref[:, PAD_ID:PAD_ID + 1]
    w2 = wp_ref[:, EOS_ID:EOS_ID + 1]
    k0 = jnp.where(stopb, p, p * w0)
    f0 = base
    slot7_k = jnp.where(stopb, 0.0, p * w2)
    slot7_f = jnp.where(stopb, BIGIDX + base + 7, base + EOS_ID)

    score_ref[...] = jnp.concatenate([sel_k, k0, slot7_k], axis=1)
    flat_ref[...] = jnp.concatenate([sel_f, f0, slot7_f], axis=1)


def _merge_kernel(score_ref, flat_ref, bs_ref, nw_ref, pi_ref):
    s = score_ref[...]
    f = flat_ref[...]
    batch = jax.lax.broadcasted_iota(jnp.int32, (BATCH, 1), 0)
    bs, nw, pi = [], [], []
    for _ in range(BEAM):
        mx = jnp.max(s, axis=1, keepdims=True)
        eq = s == mx
        mi = jnp.min(jnp.where(eq, f, jnp.int32(2 ** 31 - 1)),
                     axis=1, keepdims=True)
        sel = eq & (f == mi)
        bs.append(mx)
        nw.append(mi % VOCAB)
        pi.append(batch * BEAM + mi // VOCAB)
        s = jnp.where(sel, NEG, s)
    bs_ref[...] = jnp.concatenate(bs, axis=1)
    nw_ref[...] = jnp.concatenate(nw, axis=1)
    pi_ref[...] = jnp.concatenate(pi, axis=1)


@jax.jit
def kernel(word_prob, prob, stops, word_length):
    p2 = prob.reshape(ROWS, 1)
    st2 = stops.reshape(ROWS, 1)
    wl2 = word_length.reshape(ROWS, 1)

    grid = ROWS // ROW_BLK
    keys, flats = pl.pallas_call(
        _sel_kernel,
        grid=(grid,),
        in_specs=[
            pl.BlockSpec((ROW_BLK, 1), lambda i: (i, 0)),
            pl.BlockSpec((ROW_BLK, 1), lambda i: (i, 0)),
            pl.BlockSpec((ROW_BLK, 1), lambda i: (i, 0)),
            pl.BlockSpec((ROW_BLK, VOCAB), lambda i: (i, 0)),
        ],
        out_specs=[
            pl.BlockSpec((ROW_BLK, 8), lambda i: (i, 0)),
            pl.BlockSpec((ROW_BLK, 8), lambda i: (i, 0)),
        ],
        out_shape=[
            jax.ShapeDtypeStruct((ROWS, 8), jnp.float32),
            jax.ShapeDtypeStruct((ROWS, 8), jnp.int32),
        ],
        scratch_shapes=[
            pltpu.VMEM((ROW_BLK, NSEL), jnp.float32),
            pltpu.VMEM((ROW_BLK, NSEL), jnp.int32),
        ],
    )(p2, st2, wl2, word_prob)

    slot_is_sel = (jnp.arange(8, dtype=jnp.int32) < NSEL).astype(jnp.int32)
    addl = slot_is_sel[None, :] * (1 - st2)
    wl_c = wl2 + addl
    lp = (jnp.power((wl_c + 5).astype(jnp.float32), LPF)
          / jnp.power(jnp.float32(6.0), LPF))
    scores = jnp.log(jnp.clip(keys, 1e-20, 1.0)) / lp

    sc = scores.reshape(BATCH, BEAM * 8)
    fl = flats.reshape(BATCH, BEAM * 8)
    bs, nw, pi = pl.pallas_call(
        _merge_kernel,
        out_shape=[
            jax.ShapeDtypeStruct((BATCH, BEAM), jnp.float32),
            jax.ShapeDtypeStruct((BATCH, BEAM), jnp.int32),
            jax.ShapeDtypeStruct((BATCH, BEAM), jnp.int32),
        ],
    )(sc, fl)
    return bs, nw.reshape(-1), pi.reshape(-1)
